# Initial kernel scaffold; baseline (speedup 1.0000x reference)
import jax
import jax.numpy as jnp
from jax.experimental import pallas as pl


def kernel(x, edge_index, batch, W1, b1, W2, b2, fc1_W, fc1_b, fc2_W, fc2_b):
    raise NotImplementedError("write your pallas kernel here")


# speedup vs baseline: 7.3981x; 7.3981x over previous
import functools

import jax
import jax.numpy as jnp
from jax import lax
from jax.experimental import pallas as pl
from jax.experimental.pallas import tpu as pltpu
from jax.experimental.pallas import tpu_sc as plsc

_N = 10000
_E = 320000
_D = 128
_G = 64
_NP = 10240
_STRIPE = _NP // 16
_CHUNK = 128
_NW = 32
_CPT = 80
_EPT = _CPT * _CHUNK
_EPAD = _NW * _EPT

_MESH = plsc.VectorSubcoreMesh(core_axis_name="c", subcore_axis_name="s")


def _fill2d(ref, rows, cols, value):
    def row(i, _):
        def col(k, _):
            ref[i, pl.ds(k * 16, 16)] = jnp.full((16,), value, jnp.float32)
            return 0
        return lax.fori_loop(0, cols // 16, col, 0)
    lax.fori_loop(0, rows, row, 0)


@functools.partial(
    pl.kernel,
    out_type=jax.ShapeDtypeStruct((2 * _NP, _D), jnp.float32),
    mesh=_MESH,
    scratch_types=[
        pltpu.VMEM((_CHUNK,), jnp.int32),
        pltpu.VMEM((_CHUNK, _D), jnp.float32),
        pltpu.VMEM((_CHUNK, _D), jnp.float32),
        pltpu.VMEM_SHARED((_NP, _D), jnp.float32),
    ],
)
def _deg_pass(dst_hbm, out_hbm, dst1, ones_v, stage_v, acc):
    c = lax.axis_index("c")
    s = lax.axis_index("s")
    wid = c * 16 + s
    _fill2d(ones_v, _CHUNK, _D, 1.0)
    _fill2d(stage_v, _CHUNK, _D, 0.0)
    row0 = s * _STRIPE
    for k in range(_STRIPE // _CHUNK):
        pltpu.sync_copy(stage_v, acc.at[pl.ds(row0 + k * _CHUNK, _CHUNK)])
    plsc.subcore_barrier()
    ebase = wid * _EPT

    def body(j, _):
        pltpu.sync_copy(dst_hbm.at[pl.ds(ebase + j * _CHUNK, _CHUNK)], dst1)
        pltpu.sync_copy(ones_v, acc.at[dst1], add=True)
        return 0
    lax.fori_loop(0, _CPT, body, 0)
    plsc.subcore_barrier()
    for k in range(_STRIPE // _CHUNK):
        r = row0 + k * _CHUNK
        pltpu.sync_copy(acc.at[pl.ds(r, _CHUNK)], stage_v)
        pltpu.sync_copy(stage_v, out_hbm.at[pl.ds(c * _NP + r, _CHUNK)])


@functools.partial(
    pl.kernel,
    out_type=jax.ShapeDtypeStruct((2 * _NP, _D), jnp.float32),
    mesh=_MESH,
    scratch_types=[
        pltpu.VMEM((_CHUNK,), jnp.int32),
        pltpu.VMEM((_CHUNK,), jnp.int32),
        pltpu.VMEM((_CHUNK, _D), jnp.float32),
        pltpu.VMEM_SHARED((_NP, _D), jnp.float32),
        pltpu.SemaphoreType.DMA,
    ],
)
def _edge_pass(src_hbm, dst_hbm, table_hbm, out_hbm, src1, dst1, buf, acc, sem):
    c = lax.axis_index("c")
    s = lax.axis_index("s")
    wid = c * 16 + s
    _fill2d(buf, _CHUNK, _D, 0.0)
    row0 = s * _STRIPE
    for k in range(_STRIPE // _CHUNK):
        pltpu.sync_copy(buf, acc.at[pl.ds(row0 + k * _CHUNK, _CHUNK)])
    plsc.subcore_barrier()
    ebase = wid * _EPT

    def body(j, _):
        pltpu.sync_copy(src_hbm.at[pl.ds(ebase + j * _CHUNK, _CHUNK)], src1)
        pltpu.sync_copy(dst_hbm.at[pl.ds(ebase + j * _CHUNK, _CHUNK)], dst1)
        pltpu.async_copy(table_hbm.at[src1], buf, sem).wait()
        pltpu.sync_copy(buf, acc.at[dst1], add=True)
        return 0
    lax.fori_loop(0, _CPT, body, 0)
    plsc.subcore_barrier()
    for k in range(_STRIPE // _CHUNK):
        r = row0 + k * _CHUNK
        pltpu.sync_copy(acc.at[pl.ds(r, _CHUNK)], buf)
        pltpu.sync_copy(buf, out_hbm.at[pl.ds(c * _NP + r, _CHUNK)])


def _dis(dp):
    deg = dp[0:_N, 0:1] + dp[_NP:_NP + _N, 0:1] + 1.0
    return lax.rsqrt(deg)


def _tc1_body(x_ref, w1_ref, dp_ref, y1_ref):
    dis = _dis(dp_ref[...])
    xw = jnp.dot(x_ref[...], w1_ref[...], preferred_element_type=jnp.float32)
    y1_ref[0:_N, :] = dis * xw
    y1_ref[_N:_NP, :] = jnp.zeros((_NP - _N, _D), jnp.float32)


def _tc2_body(agg_ref, y1_ref, dp_ref, w2_ref, b1_ref, y2_ref):
    dis = _dis(dp_ref[...])
    agg = agg_ref[0:_N, :] + agg_ref[_NP:_NP + _N, :] + y1_ref[0:_N, :]
    h1 = jnp.maximum(dis * agg + b1_ref[...][None, :], 0.0)
    y2_ref[0:_N, :] = dis * jnp.dot(h1, w2_ref[...], preferred_element_type=jnp.float32)
    y2_ref[_N:_NP, :] = jnp.zeros((_NP - _N, _D), jnp.float32)


def _tc3_body(agg_ref, y2_ref, dp_ref, b2_ref, batch_ref, f1w_ref, f1b_ref,
              f2w_ref, f2b_ref, out_ref):
    dis = _dis(dp_ref[...])
    agg = agg_ref[0:_N, :] + agg_ref[_NP:_NP + _N, :] + y2_ref[0:_N, :]
    h2 = dis * agg + b2_ref[...][None, :]
    gi = lax.broadcasted_iota(jnp.int32, (_G, _N), 0)
    onehot = (batch_ref[...][None, :] == gi).astype(jnp.float32)
    ssum = jnp.dot(onehot, h2, preferred_element_type=jnp.float32)
    cnt = jnp.sum(onehot, axis=1, keepdims=True)
    p = ssum / jnp.maximum(cnt, 1.0)
    p = jnp.maximum(jnp.dot(p, f1w_ref[...], preferred_element_type=jnp.float32)
                    + f1b_ref[...][None, :], 0.0)
    out_ref[...] = (jnp.dot(p, f2w_ref[...], preferred_element_type=jnp.float32)
                    + f2b_ref[...][None, :])


_tc1 = pl.pallas_call(_tc1_body, out_shape=jax.ShapeDtypeStruct((_NP, _D), jnp.float32))
_tc2 = pl.pallas_call(_tc2_body, out_shape=jax.ShapeDtypeStruct((_NP, _D), jnp.float32))
_tc3 = pl.pallas_call(_tc3_body, out_shape=jax.ShapeDtypeStruct((_G, _D), jnp.float32))


def kernel(x, edge_index, batch, W1, b1, W2, b2, fc1_W, fc1_b, fc2_W, fc2_b):
    pad = jnp.full((_EPAD - _E,), _N, jnp.int32)
    src = jnp.concatenate([edge_index[0], pad])
    dst = jnp.concatenate([edge_index[1], pad])

    dp = _deg_pass(dst)
    y1 = _tc1(x, W1, dp)
    agg1 = _edge_pass(src, dst, y1)
    y2 = _tc2(agg1, y1, dp, W2, b1)
    agg2 = _edge_pass(src, dst, y2)
    return _tc3(agg2, y2, dp, b2, batch, fc1_W, fc1_b, fc2_W, fc2_b)


# speedup vs baseline: 9.3332x; 1.2616x over previous
import functools

import jax
import jax.numpy as jnp
from jax import lax
from jax.experimental import pallas as pl
from jax.experimental.pallas import tpu as pltpu
from jax.experimental.pallas import tpu_sc as plsc

_N = 10000
_E = 320000
_D = 128
_G = 64
_NP = 10240
_STRIPE = _NP // 16
_CHUNK = 128
_NW = 32
_CPT = 80
_EPT = _CPT * _CHUNK
_EPAD = _NW * _EPT

_MESH = plsc.VectorSubcoreMesh(core_axis_name="c", subcore_axis_name="s")


def _fill2d(ref, rows, cols, value):
    def row(i, _):
        def col(k, _):
            ref[i, pl.ds(k * 16, 16)] = jnp.full((16,), value, jnp.float32)
            return 0
        return lax.fori_loop(0, cols // 16, col, 0)
    lax.fori_loop(0, rows, row, 0)


_K = 2


@functools.partial(
    pl.kernel,
    out_type=jax.ShapeDtypeStruct((2 * _NP, _D), jnp.float32),
    mesh=_MESH,
    scratch_types=[
        [pltpu.VMEM((_CHUNK,), jnp.int32) for _ in range(_K)],
        [pltpu.SemaphoreType.DMA for _ in range(_K)],
        pltpu.VMEM((_CHUNK, _D), jnp.float32),
        pltpu.VMEM((_CHUNK, _D), jnp.float32),
        pltpu.VMEM_SHARED((_NP, _D), jnp.float32),
    ],
)
def _deg_pass(dst_hbm, out_hbm, dsts, dsems, ones_v, stage_v, acc):
    c = lax.axis_index("c")
    s = lax.axis_index("s")
    wid = c * 16 + s
    _fill2d(ones_v, _CHUNK, _D, 1.0)
    _fill2d(stage_v, _CHUNK, _D, 0.0)
    row0 = s * _STRIPE
    for k in range(_STRIPE // _CHUNK):
        pltpu.sync_copy(stage_v, acc.at[pl.ds(row0 + k * _CHUNK, _CHUNK)])
    plsc.subcore_barrier()
    ebase = wid * _EPT
    for b in range(_K):
        pltpu.async_copy(dst_hbm.at[pl.ds(ebase + b * _CHUNK, _CHUNK)],
                         dsts[b], dsems[b])

    def outer(g, _):
        for b in range(_K):
            t = g * _K + b
            pltpu.make_async_copy(
                dst_hbm.at[pl.ds(ebase + t * _CHUNK, _CHUNK)],
                dsts[b], dsems[b]).wait()
            pltpu.sync_copy(ones_v, acc.at[dsts[b]], add=True)
            tn = t + _K

            @pl.when(tn < _CPT)
            def _():
                pltpu.async_copy(dst_hbm.at[pl.ds(ebase + tn * _CHUNK, _CHUNK)],
                                 dsts[b], dsems[b])
        return 0
    lax.fori_loop(0, _CPT // _K, outer, 0)
    plsc.subcore_barrier()
    for k in range(_STRIPE // _CHUNK):
        r = row0 + k * _CHUNK
        pltpu.sync_copy(acc.at[pl.ds(r, _CHUNK)], stage_v)
        pltpu.sync_copy(stage_v, out_hbm.at[pl.ds(c * _NP + r, _CHUNK)])


_H = _K // 2


@functools.partial(
    pl.kernel,
    out_type=jax.ShapeDtypeStruct((2 * _NP, _D), jnp.float32),
    mesh=_MESH,
    scratch_types=[
        [pltpu.VMEM((_CHUNK,), jnp.int32) for _ in range(_K)],
        [pltpu.VMEM((_CHUNK,), jnp.int32) for _ in range(_K)],
        [pltpu.VMEM((_CHUNK, _D), jnp.float32) for _ in range(_K)],
        [pltpu.SemaphoreType.DMA for _ in range(_K)],
        [pltpu.SemaphoreType.DMA for _ in range(_K)],
        [pltpu.SemaphoreType.DMA for _ in range(_K)],
        pltpu.VMEM_SHARED((_NP, _D), jnp.float32),
    ],
)
def _edge_pass(src_hbm, dst_hbm, table_hbm, out_hbm,
               srcs, dsts, bufs, ssems, dsems, gsems, acc):
    c = lax.axis_index("c")
    s = lax.axis_index("s")
    wid = c * 16 + s
    _fill2d(bufs[0], _CHUNK, _D, 0.0)
    row0 = s * _STRIPE
    for k in range(_STRIPE // _CHUNK):
        pltpu.sync_copy(bufs[0], acc.at[pl.ds(row0 + k * _CHUNK, _CHUNK)])
    plsc.subcore_barrier()
    ebase = wid * _EPT

    def load(t, b):
        pltpu.async_copy(src_hbm.at[pl.ds(ebase + t * _CHUNK, _CHUNK)],
                         srcs[b], ssems[b])
        pltpu.async_copy(dst_hbm.at[pl.ds(ebase + t * _CHUNK, _CHUNK)],
                         dsts[b], dsems[b])

    def wait_load(t, b):
        pltpu.make_async_copy(src_hbm.at[pl.ds(ebase + t * _CHUNK, _CHUNK)],
                              srcs[b], ssems[b]).wait()
        pltpu.make_async_copy(dst_hbm.at[pl.ds(ebase + t * _CHUNK, _CHUNK)],
                              dsts[b], dsems[b]).wait()

    for b in range(_K):
        load(b, b)
    for b in range(_H):
        wait_load(b, b)
        pltpu.async_copy(table_hbm.at[srcs[b]], bufs[b], gsems[b])

    def outer(g, _):
        for b0 in range(_K):
            t = g * _K + b0
            tg = t + _H
            b2 = (b0 + _H) % _K

            @pl.when(tg < _CPT)
            def _():
                wait_load(tg, b2)
                pltpu.async_copy(table_hbm.at[srcs[b2]], bufs[b2], gsems[b2])

            pltpu.make_async_copy(table_hbm.at[srcs[b0]], bufs[b0],
                                  gsems[b0]).wait()
            pltpu.sync_copy(bufs[b0], acc.at[dsts[b0]], add=True)
            tn = t + _K

            @pl.when(tn < _CPT)
            def _():
                load(tn, b0)
        return 0
    lax.fori_loop(0, _CPT // _K, outer, 0)
    plsc.subcore_barrier()
    for k in range(_STRIPE // _CHUNK):
        r = row0 + k * _CHUNK
        pltpu.sync_copy(acc.at[pl.ds(r, _CHUNK)], bufs[0])
        pltpu.sync_copy(bufs[0], out_hbm.at[pl.ds(c * _NP + r, _CHUNK)])


def _dis(dp):
    deg = dp[0:_N, 0:1] + dp[_NP:_NP + _N, 0:1] + 1.0
    return lax.rsqrt(deg)


def _tc1_body(x_ref, w1_ref, dp_ref, y1_ref):
    dis = _dis(dp_ref[...])
    xw = jnp.dot(x_ref[...], w1_ref[...], preferred_element_type=jnp.float32)
    y1_ref[0:_N, :] = dis * xw
    y1_ref[_N:_NP, :] = jnp.zeros((_NP - _N, _D), jnp.float32)


def _tc2_body(agg_ref, y1_ref, dp_ref, w2_ref, b1_ref, y2_ref):
    dis = _dis(dp_ref[...])
    agg = agg_ref[0:_N, :] + agg_ref[_NP:_NP + _N, :] + y1_ref[0:_N, :]
    h1 = jnp.maximum(dis * agg + b1_ref[...][None, :], 0.0)
    y2_ref[0:_N, :] = dis * jnp.dot(h1, w2_ref[...], preferred_element_type=jnp.float32)
    y2_ref[_N:_NP, :] = jnp.zeros((_NP - _N, _D), jnp.float32)


def _tc3_body(agg_ref, y2_ref, dp_ref, b2_ref, batch_ref, f1w_ref, f1b_ref,
              f2w_ref, f2b_ref, out_ref):
    dis = _dis(dp_ref[...])
    agg = agg_ref[0:_N, :] + agg_ref[_NP:_NP + _N, :] + y2_ref[0:_N, :]
    h2 = dis * agg + b2_ref[...][None, :]
    gi = lax.broadcasted_iota(jnp.int32, (_G, _N), 0)
    onehot = (batch_ref[...][None, :] == gi).astype(jnp.float32)
    ssum = jnp.dot(onehot, h2, preferred_element_type=jnp.float32)
    cnt = jnp.sum(onehot, axis=1, keepdims=True)
    p = ssum / jnp.maximum(cnt, 1.0)
    p = jnp.maximum(jnp.dot(p, f1w_ref[...], preferred_element_type=jnp.float32)
                    + f1b_ref[...][None, :], 0.0)
    out_ref[...] = (jnp.dot(p, f2w_ref[...], preferred_element_type=jnp.float32)
                    + f2b_ref[...][None, :])


_tc1 = pl.pallas_call(_tc1_body, out_shape=jax.ShapeDtypeStruct((_NP, _D), jnp.float32))
_tc2 = pl.pallas_call(_tc2_body, out_shape=jax.ShapeDtypeStruct((_NP, _D), jnp.float32))
_tc3 = pl.pallas_call(_tc3_body, out_shape=jax.ShapeDtypeStruct((_G, _D), jnp.float32))


def kernel(x, edge_index, batch, W1, b1, W2, b2, fc1_W, fc1_b, fc2_W, fc2_b):
    pad = jnp.full((_EPAD - _E,), _N, jnp.int32)
    src = jnp.concatenate([edge_index[0], pad])
    dst = jnp.concatenate([edge_index[1], pad])

    dp = _deg_pass(dst)
    y1 = _tc1(x, W1, dp)
    agg1 = _edge_pass(src, dst, y1)
    y2 = _tc2(agg1, y1, dp, W2, b1)
    agg2 = _edge_pass(src, dst, y2)
    return _tc3(agg2, y2, dp, b2, batch, fc1_W, fc1_b, fc2_W, fc2_b)
